# baseline (speedup 1.0000x reference)
import functools

import jax
import jax.numpy as jnp
from jax import lax
from jax.experimental import pallas as pl
from jax.experimental.pallas import tpu as pltpu
from jax.experimental.pallas import tpu_sc as plsc

NW = 32
CHUNK = 2048
GPC = CHUNK // 16
H = 8


@functools.cache
def _mesh():
    return plsc.VectorSubcoreMesh(core_axis_name="c", subcore_axis_name="s")


def _sc_params():
    return pltpu.CompilerParams(needs_layout_passes=False)


def _wid():
    return lax.axis_index("s") * 2 + lax.axis_index("c")


def _zero_fill(ref, nwords):
    def body(i, _):
        ref[pl.ds(i * 16, 16)] = jnp.zeros((16,), jnp.float32)
        return 0
    lax.fori_loop(0, nwords // 16, body, 0)


def _mmc_body(x_ref, w_ref, o_ref):
    val = lax.dot_general(w_ref[...], x_ref[...], (((0,), (1,)), ((), ())),
                          preferred_element_type=jnp.float32)
    o_ref[...] = val[None]


def _mm_cpm(x, w, m_pad):
    m, c = x.shape
    h = w.shape[1]
    return pl.pallas_call(
        _mmc_body,
        grid=(pl.cdiv(m, CHUNK),),
        in_specs=[pl.BlockSpec((CHUNK, c), lambda i: (i, 0)),
                  pl.BlockSpec((c, h), lambda i: (0, 0))],
        out_specs=pl.BlockSpec((1, h, CHUNK), lambda i: (i, 0, 0)),
        out_shape=jax.ShapeDtypeStruct((m_pad // CHUNK, h, CHUNK),
                                       jnp.float32),
    )(x, w)


def _mm_body(x_ref, w_ref, o_ref):
    o_ref[...] = jnp.dot(x_ref[...], w_ref[...],
                         preferred_element_type=jnp.float32)


def _mm_small(x, w, bm=1024):
    m, c = x.shape
    h = w.shape[1]
    return pl.pallas_call(
        _mm_body,
        grid=(pl.cdiv(m, bm),),
        in_specs=[pl.BlockSpec((bm, c), lambda i: (i, 0)),
                  pl.BlockSpec((c, h), lambda i: (0, 0))],
        out_specs=pl.BlockSpec((bm, h), lambda i: (i, 0)),
        out_shape=jax.ShapeDtypeStruct((m, h), jnp.float32),
    )(x, w)


def _mlp_body(x_ref, p_ref, w1_ref, b1_ref, w2_ref, b2_ref, o_ref):
    h = jnp.dot(x_ref[...], w1_ref[...],
                preferred_element_type=jnp.float32) + b1_ref[...]
    h = jax.nn.gelu(h)
    o_ref[...] = (jnp.dot(h, w2_ref[...], preferred_element_type=jnp.float32)
                  + b2_ref[...] + p_ref[...])


def _mlp(x, prop, w1, b1, w2, b2, bm=1024):
    n, k = x.shape
    c = w2.shape[1]
    return pl.pallas_call(
        _mlp_body,
        grid=(pl.cdiv(n, bm),),
        in_specs=[pl.BlockSpec((bm, k), lambda i: (i, 0)),
                  pl.BlockSpec((bm, c), lambda i: (i, 0)),
                  pl.BlockSpec((k, c), lambda i: (0, 0)),
                  pl.BlockSpec((1, c), lambda i: (0, 0)),
                  pl.BlockSpec((c, c), lambda i: (0, 0)),
                  pl.BlockSpec((1, c), lambda i: (0, 0))],
        out_specs=pl.BlockSpec((bm, c), lambda i: (i, 0)),
        out_shape=jax.ShapeDtypeStruct((n, c), jnp.float32),
    )(x, prop, w1, b1, w2, b2)


def _read_params(params_hbm, pbuf):
    w = _wid()
    pltpu.sync_copy(params_hbm.at[pl.ds(pl.multiple_of(w * 16, 16), 16)], pbuf)
    pv = pbuf[...]
    return pv[0], pv[1], pv[2]


def _denom_body(accw, e_pad, lgt_hbm, seg_hbm, params_hbm, rec_hbm,
                pbuf, seg_v, lg_v, acc, sem):
    row0, e0, nch = _read_params(params_hbm, pbuf)
    _zero_fill(acc, accw)

    def chunk(k, _):
        base = pl.multiple_of(e0 + k * CHUNK, CHUNK)
        d1 = pltpu.async_copy(seg_hbm.at[pl.ds(base, CHUNK)], seg_v, sem)
        d2 = pltpu.async_copy(lgt_hbm.at[pl.ds(base * H, CHUNK * H)], lg_v,
                              sem)
        d1.wait()
        d2.wait()

        def group(g, _):
            segv = seg_v[pl.ds(g * 16, 16)]
            dstb = (segv - row0) * H
            mask = (dstb >= 0) & (dstb < accw)
            for c in range(H):
                lgc = lg_v[pl.ds(c * CHUNK + g * 16, 16)]
                plsc.addupdate_scatter(acc, [dstb + c], jnp.exp(lgc),
                                       mask=mask)
            return 0
        lax.fori_loop(0, GPC, group, 0)
        return 0
    lax.fori_loop(0, nch, chunk, 0)

    def recip(i, _):
        acc[pl.ds(i * 16, 16)] = 1.0 / acc[pl.ds(i * 16, 16)]
        return 0
    lax.fori_loop(0, accw // 16, recip, 0)
    pltpu.sync_copy(acc, rec_hbm.at[pl.ds(pl.multiple_of(row0 * H, 8), accw)])


def _alpha_body(ncht, e_pad, nrec, lgt_hbm, seg_hbm, rec_hbm, al_hbm,
                rec_rep, seg_v, lg_v, al_v, sem):
    w = _wid()
    pltpu.sync_copy(rec_hbm, rec_rep)

    def chunk(t, _):
        kid = w + t * NW

        @pl.when(kid < ncht)
        def _():
            base = pl.multiple_of(kid * CHUNK, CHUNK)
            d1 = pltpu.async_copy(seg_hbm.at[pl.ds(base, CHUNK)], seg_v, sem)
            d2 = pltpu.async_copy(lgt_hbm.at[pl.ds(base * H, CHUNK * H)],
                                  lg_v, sem)
            d1.wait()
            d2.wait()

            def group(g, _):
                segv = seg_v[pl.ds(g * 16, 16)]
                dstb = segv * H
                for c in range(H):
                    lgc = lg_v[pl.ds(c * CHUNK + g * 16, 16)]
                    rcc = plsc.load_gather(rec_rep, [dstb + c])
                    al_v[pl.ds(c * CHUNK + g * 16, 16)] = jnp.exp(lgc) * rcc
                return 0
            lax.fori_loop(0, GPC, group, 0)
            pltpu.sync_copy(al_v, al_hbm.at[pl.ds(base * H, CHUNK * H)])
        return 0
    lax.fori_loop(0, pl.cdiv(ncht, NW), chunk, 0)


def _kvgather_body(ncht, e_pad, al_hbm, g2idx_hbm, kv_hbm, idx_v, kv_v, sem):
    w = _wid()
    rows_per_plane = e_pad // 128

    def chunk(t, _):
        kid = w + t * NW

        @pl.when(kid < ncht)
        def _():
            base = pl.multiple_of(kid * CHUNK, CHUNK)
            ds_idx = [pltpu.async_copy(
                g2idx_hbm.at[pl.ds(
                    pl.multiple_of(c * rows_per_plane + base // 128, 16),
                    16)],
                idx_v.at[c], sem) for c in range(H)]
            for d in ds_idx:
                d.wait()
            ds_g = [pltpu.async_copy(
                al_hbm.at[idx_v.at[c].at[j]],
                kv_v.at[pl.ds(c * CHUNK + j * 128, 128)], sem)
                for c in range(H) for j in range(16)]
            for d in ds_g:
                d.wait()
            pltpu.sync_copy(kv_v, kv_hbm.at[pl.ds(base * H, CHUNK * H)])
        return 0
    lax.fori_loop(0, pl.cdiv(ncht, NW), chunk, 0)


def _prop_body(accw, e_pad, nx, x_hbm, kv_hbm, seg_hbm, g1_hbm, params_hbm,
               y_hbm, pbuf, x_rep, seg_a, g1_a, kv_a, seg_b, g1_b, kv_b, acc,
               sem_r, sem_a, sem_b):
    row0, e0, nch = _read_params(params_hbm, pbuf)
    dx = pltpu.async_copy(x_hbm, x_rep, sem_r)

    def start(k, seg_v, g1_v, kv_v, sem):
        base = pl.multiple_of(e0 + k * CHUNK, CHUNK)
        pltpu.async_copy(seg_hbm.at[pl.ds(base, CHUNK)], seg_v, sem)
        pltpu.async_copy(g1_hbm.at[pl.ds(base, CHUNK)], g1_v, sem)
        pltpu.async_copy(kv_hbm.at[pl.ds(base * H, CHUNK * H)], kv_v, sem)

    def wait(k, seg_v, g1_v, kv_v, sem):
        base = pl.multiple_of(e0 + k * CHUNK, CHUNK)
        pltpu.make_async_copy(seg_hbm.at[pl.ds(base, CHUNK)], seg_v,
                              sem).wait()
        pltpu.make_async_copy(g1_hbm.at[pl.ds(base, CHUNK)], g1_v,
                              sem).wait()
        pltpu.make_async_copy(kv_hbm.at[pl.ds(base * H, CHUNK * H)], kv_v,
                              sem).wait()

    def compute(seg_v, g1_v, kv_v):
        def group(g, _):
            segv = seg_v[pl.ds(g * 16, 16)]
            gofs = g1_v[pl.ds(g * 16, 16)] * H
            dstb = (segv - row0) * H
            mask = (dstb >= 0) & (dstb < accw)
            for c in range(H):
                kvc = kv_v[pl.ds(c * CHUNK + g * 16, 16)]
                gxc = plsc.load_gather(x_rep, [gofs + c])
                plsc.addupdate_scatter(acc, [dstb + c], kvc * gxc, mask=mask)
            return 0
        lax.fori_loop(0, GPC, group, 0)

    @pl.when(nch > 0)
    def _():
        start(0, seg_a, g1_a, kv_a, sem_a)
    _zero_fill(acc, accw)
    dx.wait()

    def pair(t, _):
        k0 = 2 * t
        k1 = k0 + 1

        @pl.when(k1 < nch)
        def _():
            start(k1, seg_b, g1_b, kv_b, sem_b)
        wait(k0, seg_a, g1_a, kv_a, sem_a)
        compute(seg_a, g1_a, kv_a)

        @pl.when(k1 < nch)
        def _():
            @pl.when(k1 + 1 < nch)
            def _():
                start(k1 + 1, seg_a, g1_a, kv_a, sem_a)
            wait(k1, seg_b, g1_b, kv_b, sem_b)
            compute(seg_b, g1_b, kv_b)
        return 0
    lax.fori_loop(0, (nch + 1) // 2, pair, 0)
    pltpu.sync_copy(acc, y_hbm.at[pl.ds(pl.multiple_of(row0 * H, 8), accw)])


def _tile_params(seg_sorted, rpt):
    bounds = jnp.searchsorted(
        seg_sorted, jnp.arange(NW + 1) * rpt).astype(jnp.int32)
    e0 = (bounds[:-1] // CHUNK) * CHUNK
    nch = (bounds[1:] - e0 + CHUNK - 1) // CHUNK
    nch = jnp.maximum(nch, 0)
    row0 = jnp.arange(NW, dtype=jnp.int32) * rpt
    params = jnp.stack([row0, e0, nch], axis=1).astype(jnp.int32)
    return jnp.pad(params, ((0, 0), (0, 13))).reshape(-1)


def kernel(prop_attr, stereo_attr, gather_idx_ijkl_jkl, gather_idx_Uijkl_ijkl,
           gather_idx_Uijkl_Uijk, gather_idx_Uijkl_ujkl, num_ijk, num_Uijk,
           W_prop, W_tri, W1, b1, W2, b2):
    n, c = prop_attr.shape
    e = stereo_attr.shape[0]
    h = W_tri.shape[1]
    l = W1.shape[0] // h - 1
    assert h == H

    rpt = ((n + NW * 8 - 1) // (NW * 8)) * 8
    npad = NW * rpt
    accw = rpt * H
    e_pad = ((e + CHUNK - 1) // CHUNK + 1) * CHUNK
    ncht = e_pad // CHUNK

    seg_ijk = jnp.minimum(gather_idx_ijkl_jkl, num_ijk - 1).astype(jnp.int32)
    seg_u = jnp.minimum(gather_idx_Uijkl_ujkl, num_Uijk - 1).astype(jnp.int32)
    pad_seg = jnp.int32(npad - 1)
    seg_ijk_p = jnp.pad(seg_ijk, (0, e_pad - e), constant_values=pad_seg)
    seg_u_p = jnp.pad(seg_u, (0, e_pad - e), constant_values=pad_seg)
    g1_p = jnp.pad(gather_idx_Uijkl_Uijk.astype(jnp.int32), (0, e_pad - e))
    g2_p = jnp.pad(gather_idx_Uijkl_ijkl.astype(jnp.int32), (0, e_pad - e))

    params_ijk = _tile_params(seg_ijk_p[:e], rpt)
    params_u = _tile_params(seg_u_p[:e], rpt)

    lgt = _mm_cpm(stereo_attr, W_tri, e_pad).reshape(-1)
    p0 = _mm_small(prop_attr, W_prop)

    rec = pl.kernel(
        functools.partial(_denom_body, accw, e_pad),
        out_type=jax.ShapeDtypeStruct((npad * H,), jnp.float32),
        mesh=_mesh(),
        compiler_params=_sc_params(),
        scratch_types=[pltpu.VMEM((16,), jnp.int32),
                       pltpu.VMEM((CHUNK,), jnp.int32),
                       pltpu.VMEM((CHUNK * H,), jnp.float32),
                       pltpu.VMEM((accw,), jnp.float32),
                       pltpu.SemaphoreType.DMA],
    )(lgt, seg_ijk_p, params_ijk)

    alt = pl.kernel(
        functools.partial(_alpha_body, ncht, e_pad, npad * H),
        out_type=jax.ShapeDtypeStruct((e_pad * H,), jnp.float32),
        mesh=_mesh(),
        compiler_params=_sc_params(),
        scratch_types=[pltpu.VMEM((npad * H,), jnp.float32),
                       pltpu.VMEM((CHUNK,), jnp.int32),
                       pltpu.VMEM((CHUNK * H,), jnp.float32),
                       pltpu.VMEM((CHUNK * H,), jnp.float32),
                       pltpu.SemaphoreType.DMA],
    )(lgt, seg_ijk_p, rec)

    g2base = (g2_p // CHUNK) * (CHUNK * H) + (g2_p % CHUNK)
    g2planes = (g2base[None, :]
                + (jnp.arange(H, dtype=jnp.int32) * CHUNK)[:, None])
    kvt = pl.kernel(
        functools.partial(_kvgather_body, ncht, e_pad),
        out_type=jax.ShapeDtypeStruct((e_pad * H,), jnp.float32),
        mesh=_mesh(),
        compiler_params=_sc_params(),
        scratch_types=[pltpu.VMEM((H, 16, 128), jnp.int32),
                       pltpu.VMEM((CHUNK * H,), jnp.float32),
                       pltpu.SemaphoreType.DMA],
    )(alt, g2planes.reshape(-1, 128))

    prop_fn = pl.kernel(
        functools.partial(_prop_body, accw, e_pad, npad * H),
        out_type=jax.ShapeDtypeStruct((npad * H,), jnp.float32),
        mesh=_mesh(),
        compiler_params=_sc_params(),
        scratch_types=[pltpu.VMEM((16,), jnp.int32),
                       pltpu.VMEM((npad * H,), jnp.float32),
                       pltpu.VMEM((CHUNK,), jnp.int32),
                       pltpu.VMEM((CHUNK,), jnp.int32),
                       pltpu.VMEM((CHUNK * H,), jnp.float32),
                       pltpu.VMEM((CHUNK,), jnp.int32),
                       pltpu.VMEM((CHUNK,), jnp.int32),
                       pltpu.VMEM((CHUNK * H,), jnp.float32),
                       pltpu.VMEM((accw,), jnp.float32),
                       pltpu.SemaphoreType.DMA,
                       pltpu.SemaphoreType.DMA,
                       pltpu.SemaphoreType.DMA],
    )
    x = jnp.pad(p0, ((0, npad - n), (0, 0))).reshape(-1)
    outs = [p0]
    for _ in range(l):
        x = prop_fn(x, kvt, seg_u_p, g1_p, params_u)
        outs.append(x.reshape(npad, H)[:n])

    new_prop = jnp.concatenate(outs, axis=-1)
    return _mlp(new_prop, prop_attr, W1, b1.reshape(1, -1), W2,
                b2.reshape(1, -1))


# speedup vs baseline: 1.2803x; 1.2803x over previous
import functools

import jax
import jax.numpy as jnp
from jax import lax
from jax.experimental import pallas as pl
from jax.experimental.pallas import tpu as pltpu
from jax.experimental.pallas import tpu_sc as plsc

NW = 32
CHUNK = 1024
GPC = CHUNK // 16
H = 8


@functools.cache
def _mesh():
    return plsc.VectorSubcoreMesh(core_axis_name="c", subcore_axis_name="s")


def _sc_params():
    return pltpu.CompilerParams(needs_layout_passes=False)


def _wid():
    return lax.axis_index("s") * 2 + lax.axis_index("c")


def _zero_fill(ref, nwords):
    def body(i, _):
        ref[pl.ds(i * 16, 16)] = jnp.zeros((16,), jnp.float32)
        return 0
    lax.fori_loop(0, nwords // 16, body, 0)


def _mmc_body(x_ref, w_ref, o_ref):
    val = lax.dot_general(w_ref[...], x_ref[...], (((0,), (1,)), ((), ())),
                          preferred_element_type=jnp.float32)
    o_ref[...] = val[None]


def _mm_cpm(x, w, m_pad):
    m, c = x.shape
    h = w.shape[1]
    return pl.pallas_call(
        _mmc_body,
        grid=(pl.cdiv(m, CHUNK),),
        in_specs=[pl.BlockSpec((CHUNK, c), lambda i: (i, 0)),
                  pl.BlockSpec((c, h), lambda i: (0, 0))],
        out_specs=pl.BlockSpec((1, h, CHUNK), lambda i: (i, 0, 0)),
        out_shape=jax.ShapeDtypeStruct((m_pad // CHUNK, h, CHUNK),
                                       jnp.float32),
    )(x, w)


def _mm_body(x_ref, w_ref, o_ref):
    o_ref[...] = jnp.dot(x_ref[...], w_ref[...],
                         preferred_element_type=jnp.float32)


def _mm_small(x, w, bm=1024):
    m, c = x.shape
    h = w.shape[1]
    return pl.pallas_call(
        _mm_body,
        grid=(pl.cdiv(m, bm),),
        in_specs=[pl.BlockSpec((bm, c), lambda i: (i, 0)),
                  pl.BlockSpec((c, h), lambda i: (0, 0))],
        out_specs=pl.BlockSpec((bm, h), lambda i: (i, 0)),
        out_shape=jax.ShapeDtypeStruct((m, h), jnp.float32),
    )(x, w)


def _mlp_body(x_ref, p_ref, w1_ref, b1_ref, w2_ref, b2_ref, o_ref):
    h = jnp.dot(x_ref[...], w1_ref[...],
                preferred_element_type=jnp.float32) + b1_ref[...]
    h = jax.nn.gelu(h)
    o_ref[...] = (jnp.dot(h, w2_ref[...], preferred_element_type=jnp.float32)
                  + b2_ref[...] + p_ref[...])


def _mlp(x, prop, w1, b1, w2, b2, bm=1024):
    n, k = x.shape
    c = w2.shape[1]
    return pl.pallas_call(
        _mlp_body,
        grid=(pl.cdiv(n, bm),),
        in_specs=[pl.BlockSpec((bm, k), lambda i: (i, 0)),
                  pl.BlockSpec((bm, c), lambda i: (i, 0)),
                  pl.BlockSpec((k, c), lambda i: (0, 0)),
                  pl.BlockSpec((1, c), lambda i: (0, 0)),
                  pl.BlockSpec((c, c), lambda i: (0, 0)),
                  pl.BlockSpec((1, c), lambda i: (0, 0))],
        out_specs=pl.BlockSpec((bm, c), lambda i: (i, 0)),
        out_shape=jax.ShapeDtypeStruct((n, c), jnp.float32),
    )(x, prop, w1, b1, w2, b2)


def _read_params(params_hbm, pbuf):
    w = _wid()
    pltpu.sync_copy(params_hbm.at[pl.ds(pl.multiple_of(w * 16, 16), 16)], pbuf)
    pv = pbuf[...]
    return pv[0], pv[1], pv[2]


def _denom_body(accw, e_pad, lgt_hbm, seg_hbm, params_hbm, rec_hbm,
                pbuf, seg_v, lg_v, acc, sem):
    row0, e0, nch = _read_params(params_hbm, pbuf)
    _zero_fill(acc, accw)

    def chunk(k, _):
        base = pl.multiple_of(e0 + k * CHUNK, CHUNK)
        d1 = pltpu.async_copy(seg_hbm.at[pl.ds(base, CHUNK)], seg_v, sem)
        d2 = pltpu.async_copy(lgt_hbm.at[pl.ds(base * H, CHUNK * H)], lg_v,
                              sem)
        d1.wait()
        d2.wait()

        def group(g, _):
            segv = seg_v[pl.ds(g * 16, 16)]
            dstb = (segv - row0) * H
            mask = (dstb >= 0) & (dstb < accw)
            for c in range(H):
                lgc = lg_v[pl.ds(c * CHUNK + g * 16, 16)]
                plsc.addupdate_scatter(acc, [dstb + c], jnp.exp(lgc),
                                       mask=mask)
            return 0
        plsc.parallel_loop(0, GPC, 1, unroll=4, carry=None)(
            lambda g: group(g, 0))
        return 0
    lax.fori_loop(0, nch, chunk, 0)

    def recip(i, _):
        acc[pl.ds(i * 16, 16)] = 1.0 / acc[pl.ds(i * 16, 16)]
        return 0
    lax.fori_loop(0, accw // 16, recip, 0)
    pltpu.sync_copy(acc, rec_hbm.at[pl.ds(pl.multiple_of(row0 * H, 8), accw)])


def _alpha_body(ncht, e_pad, nrec, lgt_hbm, seg_hbm, rec_hbm, al_hbm,
                rec_rep, seg_v, lg_v, al_v, sem):
    w = _wid()
    pltpu.sync_copy(rec_hbm, rec_rep)

    def chunk(t, _):
        kid = w + t * NW

        @pl.when(kid < ncht)
        def _():
            base = pl.multiple_of(kid * CHUNK, CHUNK)
            d1 = pltpu.async_copy(seg_hbm.at[pl.ds(base, CHUNK)], seg_v, sem)
            d2 = pltpu.async_copy(lgt_hbm.at[pl.ds(base * H, CHUNK * H)],
                                  lg_v, sem)
            d1.wait()
            d2.wait()

            def group(g, _):
                segv = seg_v[pl.ds(g * 16, 16)]
                dstb = segv * H
                for c in range(H):
                    lgc = lg_v[pl.ds(c * CHUNK + g * 16, 16)]
                    rcc = plsc.load_gather(rec_rep, [dstb + c])
                    al_v[pl.ds(c * CHUNK + g * 16, 16)] = jnp.exp(lgc) * rcc
                return 0
            plsc.parallel_loop(0, GPC, 1, unroll=4, carry=None)(
                lambda g: group(g, 0))
            pltpu.sync_copy(al_v, al_hbm.at[pl.ds(base * H, CHUNK * H)])
        return 0
    lax.fori_loop(0, pl.cdiv(ncht, NW), chunk, 0)


def _kvgather_body(ncht, e_pad, al_hbm, g2idx_hbm, kv_hbm, idx_v, kv_v, sem):
    w = _wid()
    rows_per_plane = e_pad // 128

    def chunk(t, _):
        kid = w + t * NW

        @pl.when(kid < ncht)
        def _():
            base = pl.multiple_of(kid * CHUNK, CHUNK)
            ds_idx = [pltpu.async_copy(
                g2idx_hbm.at[pl.ds(
                    pl.multiple_of(c * rows_per_plane + base // 128, 8), 8)],
                idx_v.at[c], sem) for c in range(H)]
            for d in ds_idx:
                d.wait()
            ds_g = [pltpu.async_copy(
                al_hbm.at[idx_v.at[c].at[j]],
                kv_v.at[pl.ds(c * CHUNK + j * 128, 128)], sem)
                for c in range(H) for j in range(8)]
            for d in ds_g:
                d.wait()
            pltpu.sync_copy(kv_v, kv_hbm.at[pl.ds(base * H, CHUNK * H)])
        return 0
    lax.fori_loop(0, pl.cdiv(ncht, NW), chunk, 0)


def _prop_body(accw, e_pad, nx, x_hbm, kv_hbm, seg_hbm, g1_hbm, params_hbm,
               y_hbm, pbuf, x_rep, seg_a, g1_a, kv_a, seg_b, g1_b, kv_b, acc,
               sem_r, sem_a, sem_b):
    row0, e0, nch = _read_params(params_hbm, pbuf)
    dx = pltpu.async_copy(x_hbm, x_rep, sem_r)

    def start(k, seg_v, g1_v, kv_v, sem):
        base = pl.multiple_of(e0 + k * CHUNK, CHUNK)
        pltpu.async_copy(seg_hbm.at[pl.ds(base, CHUNK)], seg_v, sem)
        pltpu.async_copy(g1_hbm.at[pl.ds(base, CHUNK)], g1_v, sem)
        pltpu.async_copy(kv_hbm.at[pl.ds(base * H, CHUNK * H)], kv_v, sem)

    def wait(k, seg_v, g1_v, kv_v, sem):
        base = pl.multiple_of(e0 + k * CHUNK, CHUNK)
        pltpu.make_async_copy(seg_hbm.at[pl.ds(base, CHUNK)], seg_v,
                              sem).wait()
        pltpu.make_async_copy(g1_hbm.at[pl.ds(base, CHUNK)], g1_v,
                              sem).wait()
        pltpu.make_async_copy(kv_hbm.at[pl.ds(base * H, CHUNK * H)], kv_v,
                              sem).wait()

    def compute(seg_v, g1_v, kv_v):
        def group(g, _):
            segv = seg_v[pl.ds(g * 16, 16)]
            gofs = g1_v[pl.ds(g * 16, 16)] * H
            dstb = (segv - row0) * H
            mask = (dstb >= 0) & (dstb < accw)
            for c in range(H):
                kvc = kv_v[pl.ds(c * CHUNK + g * 16, 16)]
                gxc = plsc.load_gather(x_rep, [gofs + c])
                plsc.addupdate_scatter(acc, [dstb + c], kvc * gxc, mask=mask)
            return 0
        plsc.parallel_loop(0, GPC, 1, unroll=4, carry=None)(
            lambda g: group(g, 0))

    @pl.when(nch > 0)
    def _():
        start(0, seg_a, g1_a, kv_a, sem_a)
    _zero_fill(acc, accw)
    dx.wait()

    def pair(t, _):
        k0 = 2 * t
        k1 = k0 + 1

        @pl.when(k1 < nch)
        def _():
            start(k1, seg_b, g1_b, kv_b, sem_b)
        wait(k0, seg_a, g1_a, kv_a, sem_a)
        compute(seg_a, g1_a, kv_a)

        @pl.when(k1 < nch)
        def _():
            @pl.when(k1 + 1 < nch)
            def _():
                start(k1 + 1, seg_a, g1_a, kv_a, sem_a)
            wait(k1, seg_b, g1_b, kv_b, sem_b)
            compute(seg_b, g1_b, kv_b)
        return 0
    lax.fori_loop(0, (nch + 1) // 2, pair, 0)
    pltpu.sync_copy(acc, y_hbm.at[pl.ds(pl.multiple_of(row0 * H, 8), accw)])


def _tile_params(seg_sorted, rpt):
    bounds = jnp.searchsorted(
        seg_sorted, jnp.arange(NW + 1) * rpt).astype(jnp.int32)
    e0 = (bounds[:-1] // CHUNK) * CHUNK
    nch = (bounds[1:] - e0 + CHUNK - 1) // CHUNK
    nch = jnp.maximum(nch, 0)
    row0 = jnp.arange(NW, dtype=jnp.int32) * rpt
    params = jnp.stack([row0, e0, nch], axis=1).astype(jnp.int32)
    return jnp.pad(params, ((0, 0), (0, 13))).reshape(-1)


def kernel(prop_attr, stereo_attr, gather_idx_ijkl_jkl, gather_idx_Uijkl_ijkl,
           gather_idx_Uijkl_Uijk, gather_idx_Uijkl_ujkl, num_ijk, num_Uijk,
           W_prop, W_tri, W1, b1, W2, b2):
    n, c = prop_attr.shape
    e = stereo_attr.shape[0]
    h = W_tri.shape[1]
    l = W1.shape[0] // h - 1
    assert h == H

    rpt = ((n + NW * 8 - 1) // (NW * 8)) * 8
    npad = NW * rpt
    accw = rpt * H
    e_pad = ((e + CHUNK - 1) // CHUNK + 1) * CHUNK
    ncht = e_pad // CHUNK

    seg_ijk = jnp.minimum(gather_idx_ijkl_jkl, num_ijk - 1).astype(jnp.int32)
    seg_u = jnp.minimum(gather_idx_Uijkl_ujkl, num_Uijk - 1).astype(jnp.int32)
    pad_seg = jnp.int32(npad - 1)
    seg_ijk_p = jnp.pad(seg_ijk, (0, e_pad - e), constant_values=pad_seg)
    seg_u_p = jnp.pad(seg_u, (0, e_pad - e), constant_values=pad_seg)
    g1_p = jnp.pad(gather_idx_Uijkl_Uijk.astype(jnp.int32), (0, e_pad - e))
    g2_p = jnp.pad(gather_idx_Uijkl_ijkl.astype(jnp.int32), (0, e_pad - e))

    params_ijk = _tile_params(seg_ijk_p[:e], rpt)
    params_u = _tile_params(seg_u_p[:e], rpt)

    lgt = _mm_cpm(stereo_attr, W_tri, e_pad).reshape(-1)
    p0 = _mm_small(prop_attr, W_prop)

    rec = pl.kernel(
        functools.partial(_denom_body, accw, e_pad),
        out_type=jax.ShapeDtypeStruct((npad * H,), jnp.float32),
        mesh=_mesh(),
        compiler_params=_sc_params(),
        scratch_types=[pltpu.VMEM((16,), jnp.int32),
                       pltpu.VMEM((CHUNK,), jnp.int32),
                       pltpu.VMEM((CHUNK * H,), jnp.float32),
                       pltpu.VMEM((accw,), jnp.float32),
                       pltpu.SemaphoreType.DMA],
    )(lgt, seg_ijk_p, params_ijk)

    alt = pl.kernel(
        functools.partial(_alpha_body, ncht, e_pad, npad * H),
        out_type=jax.ShapeDtypeStruct((e_pad * H,), jnp.float32),
        mesh=_mesh(),
        compiler_params=_sc_params(),
        scratch_types=[pltpu.VMEM((npad * H,), jnp.float32),
                       pltpu.VMEM((CHUNK,), jnp.int32),
                       pltpu.VMEM((CHUNK * H,), jnp.float32),
                       pltpu.VMEM((CHUNK * H,), jnp.float32),
                       pltpu.SemaphoreType.DMA],
    )(lgt, seg_ijk_p, rec)

    g2base = (g2_p // CHUNK) * (CHUNK * H) + (g2_p % CHUNK)
    g2planes = (g2base[None, :]
                + (jnp.arange(H, dtype=jnp.int32) * CHUNK)[:, None])
    kvt = pl.kernel(
        functools.partial(_kvgather_body, ncht, e_pad),
        out_type=jax.ShapeDtypeStruct((e_pad * H,), jnp.float32),
        mesh=_mesh(),
        compiler_params=_sc_params(),
        scratch_types=[pltpu.VMEM((H, 8, 128), jnp.int32),
                       pltpu.VMEM((CHUNK * H,), jnp.float32),
                       pltpu.SemaphoreType.DMA],
    )(alt, g2planes.reshape(-1, 128))

    prop_fn = pl.kernel(
        functools.partial(_prop_body, accw, e_pad, npad * H),
        out_type=jax.ShapeDtypeStruct((npad * H,), jnp.float32),
        mesh=_mesh(),
        compiler_params=_sc_params(),
        scratch_types=[pltpu.VMEM((16,), jnp.int32),
                       pltpu.VMEM((npad * H,), jnp.float32),
                       pltpu.VMEM((CHUNK,), jnp.int32),
                       pltpu.VMEM((CHUNK,), jnp.int32),
                       pltpu.VMEM((CHUNK * H,), jnp.float32),
                       pltpu.VMEM((CHUNK,), jnp.int32),
                       pltpu.VMEM((CHUNK,), jnp.int32),
                       pltpu.VMEM((CHUNK * H,), jnp.float32),
                       pltpu.VMEM((accw,), jnp.float32),
                       pltpu.SemaphoreType.DMA,
                       pltpu.SemaphoreType.DMA,
                       pltpu.SemaphoreType.DMA],
    )
    x = jnp.pad(p0, ((0, npad - n), (0, 0))).reshape(-1)
    outs = [p0]
    for _ in range(l):
        x = prop_fn(x, kvt, seg_u_p, g1_p, params_u)
        outs.append(x.reshape(npad, H)[:n])

    new_prop = jnp.concatenate(outs, axis=-1)
    return _mlp(new_prop, prop_attr, W1, b1.reshape(1, -1), W2,
                b2.reshape(1, -1))
